# Initial kernel scaffold; baseline (speedup 1.0000x reference)
import jax
import jax.numpy as jnp
from jax.experimental import pallas as pl


def kernel(x, edge_index, batch, W_rel_0, b_rel_0, W_root_0, W_rel_1, b_rel_1, W_root_1, W_rel_2, b_rel_2, W_root_2, W1, b1, W2, b2):
    raise NotImplementedError("write your pallas kernel here")


# speedup vs baseline: 4.5251x; 4.5251x over previous
import functools

import jax
import jax.numpy as jnp
from jax import lax
from jax.experimental import pallas as pl
from jax.experimental.pallas import tpu as pltpu
from jax.experimental.pallas import tpu_sc as plsc

NC = 2
NS = 16
NW = NC * NS
CHUNK = 128
N = 10000
NP = 10240
D = 128
B = 64
ROWS_PER_SUB = NP // NS

_F32 = jnp.float32
_HIGH = lax.Precision.DEFAULT


def _mm_t(a, w):
    return lax.dot_general(a, w, dimension_numbers=(((1,), (1,)), ((), ())),
                           precision=_HIGH, preferred_element_type=_F32)


def _sc_segment_sum(m, src3, dst3, zrows, n_chunks):
    mesh = plsc.VectorSubcoreMesh(core_axis_name="c", subcore_axis_name="s")

    @functools.partial(
        pl.kernel,
        out_type=jax.ShapeDtypeStruct((NC, NP, D), _F32),
        mesh=mesh,
        scratch_types=[
            pltpu.VMEM((n_chunks, CHUNK), jnp.int32),
            pltpu.VMEM((n_chunks, CHUNK), jnp.int32),
            pltpu.VMEM((CHUNK, D), _F32),
            pltpu.VMEM_SHARED((NP, D), _F32),
            pltpu.SemaphoreType.DMA,
        ],
    )
    def k(m_hbm, src_hbm, dst_hbm, z_hbm, out_hbm, src_v, dst_v, rows_v,
          acc_sh, sem):
        cid = lax.axis_index("c")
        sid = lax.axis_index("s")
        wid = sid * NC + cid
        pltpu.sync_copy(src_hbm.at[wid], src_v)
        pltpu.sync_copy(dst_hbm.at[wid], dst_v)
        pltpu.sync_copy(z_hbm, acc_sh.at[pl.ds(sid * ROWS_PER_SUB, ROWS_PER_SUB)])
        plsc.subcore_barrier()

        @pl.loop(0, n_chunks)
        def _(j):
            pltpu.async_copy(m_hbm.at[src_v.at[j]], rows_v, sem).wait()
            pltpu.sync_copy(rows_v, acc_sh.at[dst_v.at[j]], add=True)

        plsc.subcore_barrier()
        pltpu.sync_copy(
            acc_sh.at[pl.ds(sid * ROWS_PER_SUB, ROWS_PER_SUB)],
            out_hbm.at[cid].at[pl.ds(sid * ROWS_PER_SUB, ROWS_PER_SUB)])

    return k(m, src3, dst3, zrows)


def _tc_pre(h, wr, br, wt):
    def body(h_ref, wr_ref, br_ref, wt_ref, m_ref, r_ref):
        hv = h_ref[...]
        m_ref[...] = _mm_t(hv, wr_ref[...])
        r_ref[...] = _mm_t(hv, wt_ref[...]) + br_ref[...]

    return pl.pallas_call(
        body,
        out_shape=(jax.ShapeDtypeStruct((NP, D), _F32),
                   jax.ShapeDtypeStruct((NP, D), _F32)),
    )(h, wr, br.reshape(1, D), wt)


def _tc_mid(acc, r, wr, br, wt):
    def body(acc_ref, r_ref, wr_ref, br_ref, wt_ref, m_ref, ro_ref):
        h = jnp.maximum(acc_ref[0] + acc_ref[1] + r_ref[...], 0.0)
        m_ref[...] = _mm_t(h, wr_ref[...])
        ro_ref[...] = _mm_t(h, wt_ref[...]) + br_ref[...]

    return pl.pallas_call(
        body,
        out_shape=(jax.ShapeDtypeStruct((NP, D), _F32),
                   jax.ShapeDtypeStruct((NP, D), _F32)),
    )(acc, r, wr, br.reshape(1, D), wt)


def _tc_final(acc, r, batch_row, w1, b1, w2, b2):
    def body(acc_ref, r_ref, b_ref, w1_ref, b1_ref, w2_ref, b2_ref, y_ref):
        h = jnp.maximum(acc_ref[0] + acc_ref[1] + r_ref[...], 0.0)
        seg = b_ref[...]
        mask = (lax.broadcasted_iota(jnp.int32, (B, NP), 0) == seg)
        pooled = lax.dot_general(mask.astype(_F32), h,
                                 dimension_numbers=(((1,), (0,)), ((), ())),
                                 precision=_HIGH, preferred_element_type=_F32)
        t = jnp.maximum(_mm_t(pooled, w1_ref[...]) + b1_ref[...], 0.0)
        y_ref[...] = jnp.sum(t * w2_ref[...], axis=1, keepdims=True) + b2_ref[...]

    return pl.pallas_call(
        body,
        out_shape=jax.ShapeDtypeStruct((B, 1), _F32),
    )(acc, r, batch_row, w1, b1.reshape(1, D), w2, b2.reshape(1, 1))


def kernel(x, edge_index, batch,
           W_rel_0, b_rel_0, W_root_0,
           W_rel_1, b_rel_1, W_root_1,
           W_rel_2, b_rel_2, W_root_2,
           W1, b1, W2, b2):
    e = edge_index.shape[1]
    n_chunks = -(-e // (NW * CHUNK))
    e_pad = NW * n_chunks * CHUNK

    src = edge_index[0].astype(jnp.int32)
    dst = edge_index[1].astype(jnp.int32)
    src3 = jnp.concatenate(
        [src, jnp.zeros((e_pad - e,), jnp.int32)]).reshape(NW, n_chunks, CHUNK)
    dst3 = jnp.concatenate(
        [dst, jnp.full((e_pad - e,), N, jnp.int32)]).reshape(NW, n_chunks, CHUNK)
    batch_row = jnp.concatenate(
        [batch.astype(jnp.int32), jnp.full((NP - N,), B, jnp.int32)]
    ).reshape(1, NP)
    xp = jnp.concatenate([x, jnp.zeros((NP - N, D), _F32)], axis=0)
    zrows = jnp.zeros((ROWS_PER_SUB, D), _F32)

    m, r = _tc_pre(xp, W_rel_0, b_rel_0, W_root_0)
    acc = _sc_segment_sum(m, src3, dst3, zrows, n_chunks)
    m, r = _tc_mid(acc, r, W_rel_1, b_rel_1, W_root_1)
    acc = _sc_segment_sum(m, src3, dst3, zrows, n_chunks)
    m, r = _tc_mid(acc, r, W_rel_2, b_rel_2, W_root_2)
    acc = _sc_segment_sum(m, src3, dst3, zrows, n_chunks)
    return _tc_final(acc, r, batch_row, W1, b1, W2, b2)
